# baseline (speedup 1.0000x reference)
import functools

import jax
import jax.numpy as jnp
from jax import lax
from jax.experimental import pallas as pl
from jax.experimental.pallas import tpu as pltpu
from jax.experimental.pallas import tpu_sc as plsc

N = 10000
E = 320000
EP = 327680
LAN = 16
D = 128
NC = 2
NS = 16
EDG = EP // (NC * NS)
CH = 32
NB = 4
NG = EDG // (NB * CH)
ND = 10240
RPD = ND // NS


def _sc_agg_body(x_hbm, g2_hbm, zeros_hbm, acc_out,
                 src_f, dst_f, rows, acc_sh, gsem, ssem):
    c = lax.axis_index("c")
    s = lax.axis_index("s")
    w = c * NS + s

    pltpu.sync_copy(zeros_hbm, acc_sh.at[pl.ds(s * RPD, RPD)])
    pltpu.sync_copy(g2_hbm.at[0, pl.ds(w * EDG, EDG)], src_f)
    pltpu.sync_copy(g2_hbm.at[1, pl.ds(w * EDG, EDG)], dst_f)
    plsc.subcore_barrier()

    nsub = CH // LAN

    def drain_scatters():
        for _ in range(NB * nsub):
            pltpu.make_async_copy(rows.at[0, pl.ds(0, LAN)],
                                  acc_sh.at[pl.ds(0, LAN)], ssem).wait()

    def group(j, carry):
        @pl.when(j > 0)
        def _():
            drain_scatters()
        gh = []
        for b in range(NB):
            off = (j * NB + b) * CH
            gh.append(pltpu.async_copy(x_hbm.at[src_f.at[pl.ds(off, CH)]],
                                       rows.at[b], gsem))
        for b in range(NB):
            gh[b].wait()
            off = (j * NB + b) * CH
            for k in range(nsub):
                iv = dst_f[pl.ds(off + k * LAN, LAN)]
                pltpu.async_copy(rows.at[b, pl.ds(k * LAN, LAN)],
                                 acc_sh.at[iv], ssem, add=True)
        return carry

    lax.fori_loop(0, NG, group, 0)
    drain_scatters()
    plsc.subcore_barrier()

    pltpu.sync_copy(acc_sh.at[pl.ds(s * RPD, RPD)],
                    acc_out.at[c, pl.ds(s * RPD, RPD)])


def _sc_deg_body(g2_hbm, zeros_hbm, ones_hbm, deg_out,
                 dst_f, ones_v, deg_sh, dsem):
    c = lax.axis_index("c")
    s = lax.axis_index("s")
    w = c * NS + s

    pltpu.sync_copy(zeros_hbm, deg_sh.at[pl.ds(s * RPD, RPD)])
    pltpu.sync_copy(ones_hbm, ones_v)
    pltpu.sync_copy(g2_hbm.at[1, pl.ds(w * EDG, EDG)], dst_f)
    plsc.subcore_barrier()

    def group(j, carry):
        sh = []
        for k in range(8):
            iv = dst_f[pl.ds((j * 8 + k) * LAN, LAN)]
            sh.append(pltpu.async_copy(ones_v, deg_sh.at[iv],
                                       dsem, add=True))
        for h in sh:
            h.wait()
        return carry

    lax.fori_loop(0, EDG // (8 * LAN), group, 0)
    plsc.subcore_barrier()

    pltpu.sync_copy(deg_sh.at[pl.ds(s * RPD, RPD)],
                    deg_out.at[c, pl.ds(s * RPD, RPD)])


@functools.lru_cache(maxsize=1)
def _sc_kernels():
    mesh = plsc.VectorSubcoreMesh(core_axis_name="c", subcore_axis_name="s")
    sc_agg = pl.kernel(
        _sc_agg_body,
        out_type=jax.ShapeDtypeStruct((NC, ND, D), jnp.float32),
        mesh=mesh,
        scratch_types=[
            pltpu.VMEM((EDG,), jnp.int32),
            pltpu.VMEM((EDG,), jnp.int32),
            pltpu.VMEM((NB, CH, D), jnp.float32),
            pltpu.VMEM_SHARED((ND, D), jnp.float32),
            pltpu.SemaphoreType.DMA,
            pltpu.SemaphoreType.DMA,
        ],
        name="sage_sc_agg",
    )
    sc_deg = pl.kernel(
        _sc_deg_body,
        out_type=jax.ShapeDtypeStruct((NC, ND, D), jnp.float32),
        mesh=mesh,
        scratch_types=[
            pltpu.VMEM((EDG,), jnp.int32),
            pltpu.VMEM((LAN, D), jnp.float32),
            pltpu.VMEM_SHARED((ND, D), jnp.float32),
            pltpu.SemaphoreType.DMA,
        ],
        name="sage_sc_deg",
    )
    return sc_agg, sc_deg


_ROWS = 200
_TCG = N // _ROWS


def _neigh(acc_ref, deg_ref, wn_ref, y_ref):
    agg = acc_ref[0] + acc_ref[1]
    dg = jnp.maximum(deg_ref[0, :, 0:1] + deg_ref[1, :, 0:1], 1.0)
    hn = agg / dg
    return y_ref[...] + jnp.dot(hn, wn_ref[...],
                                preferred_element_type=jnp.float32)


def _tca_body(h_ref, ws_ref, b_ref, o_ref):
    o_ref[...] = jnp.dot(h_ref[...], ws_ref[...],
                         preferred_element_type=jnp.float32) + b_ref[...]


def _tc1b_body(acc_ref, deg_ref, y_ref, wn_ref, o_ref):
    o_ref[...] = jnp.maximum(_neigh(acc_ref, deg_ref, wn_ref, y_ref), 0.0)


def _tc2b_body(acc_ref, deg_ref, y_ref, wn_ref,
               fcw_ref, fcb_ref, fc2w_ref, fc2b_ref, o_ref):
    t = jnp.maximum(_neigh(acc_ref, deg_ref, wn_ref, y_ref), 0.0)
    t = jnp.dot(t, fcw_ref[...], preferred_element_type=jnp.float32) + fcb_ref[...]
    t = jnp.maximum(t, 0.0)
    t = jnp.dot(t, fc2w_ref[...], preferred_element_type=jnp.float32) + fc2b_ref[...]
    o_ref[...] = jnp.maximum(t, 0.0)


def _tc3b_body(acc_ref, deg_ref, y_ref, wn_ref, o_ref):
    o_ref[...] = _neigh(acc_ref, deg_ref, wn_ref, y_ref)


_RSPEC = pl.BlockSpec((_ROWS, D), lambda i: (i, 0))
_WSPEC = pl.BlockSpec((D, D), lambda i: (0, 0))
_BSPEC = pl.BlockSpec((1, D), lambda i: (0, 0))
_ASPEC = pl.BlockSpec((NC, _ROWS, D), lambda i: (0, i, 0))


def _tc_call(body, specs):
    return pl.pallas_call(
        body,
        grid=(_TCG,),
        in_specs=specs,
        out_specs=_RSPEC,
        out_shape=jax.ShapeDtypeStruct((N, D), jnp.float32),
    )


def _tca(h, wsT, b):
    return _tc_call(_tca_body, [_RSPEC, _WSPEC, _BSPEC])(h, wsT, b)


def kernel(g, x, Ws1, Wn1, b1, Ws2, Wn2, b2, fcW, fcb, fc2W, fc2b, Ws3, Wn3, b3):
    npad = EP - E
    pad = jnp.stack([
        jnp.arange(npad, dtype=jnp.int32) % N,
        N + (jnp.arange(npad, dtype=jnp.int32) % (ND - N)),
    ])
    g2 = jnp.concatenate([g, pad], axis=1)
    zeros = jnp.zeros((RPD, D), jnp.float32)
    ones = jnp.ones((LAN, D), jnp.float32)

    sc_agg, sc_deg = _sc_kernels()
    deg = sc_deg(g2, zeros, ones)
    acc1 = sc_agg(x, g2, zeros)
    y1 = _tca(x, Ws1.T, b1.reshape(1, D))
    h1 = _tc_call(_tc1b_body, [_ASPEC, _ASPEC, _RSPEC, _WSPEC])(
        acc1, deg, y1, Wn1.T)

    acc2 = sc_agg(h1, g2, zeros)
    y2 = _tca(h1, Ws2.T, b2.reshape(1, D))
    h2 = _tc_call(_tc2b_body,
                  [_ASPEC, _ASPEC, _RSPEC, _WSPEC,
                   _WSPEC, _BSPEC, _WSPEC, _BSPEC])(
        acc2, deg, y2, Wn2.T,
        fcW.T, fcb.reshape(1, D), fc2W.T, fc2b.reshape(1, D))

    acc3 = sc_agg(h2, g2, zeros)
    y3 = _tca(h2, Ws3.T, b3.reshape(1, D))
    out = _tc_call(_tc3b_body, [_ASPEC, _ASPEC, _RSPEC, _WSPEC])(
        acc3, deg, y3, Wn3.T)
    return out


# speedup vs baseline: 1.0577x; 1.0577x over previous
import functools

import jax
import jax.numpy as jnp
from jax import lax
from jax.experimental import pallas as pl
from jax.experimental.pallas import tpu as pltpu
from jax.experimental.pallas import tpu_sc as plsc

N = 10000
E = 320000
EP = 327680
LAN = 16
D = 128
NC = 2
NS = 16
EDG = EP // (NC * NS)
CH = 32
NB = 5
NG = EDG // (NB * CH)
ND = 10240
RPD = ND // NS


def _sc_agg_body(x_hbm, g2_hbm, zeros_hbm, acc_out,
                 src_f, dst_f, rows, acc_sh, gsem, ssem):
    c = lax.axis_index("c")
    s = lax.axis_index("s")
    w = c * NS + s

    pltpu.sync_copy(zeros_hbm, acc_sh.at[pl.ds(s * RPD, RPD)])
    pltpu.sync_copy(g2_hbm.at[0, pl.ds(w * EDG, EDG)], src_f)
    pltpu.sync_copy(g2_hbm.at[1, pl.ds(w * EDG, EDG)], dst_f)
    plsc.subcore_barrier()

    nsub = CH // LAN

    def drain_scatters():
        for _ in range(NB * nsub):
            pltpu.make_async_copy(rows.at[0, pl.ds(0, LAN)],
                                  acc_sh.at[pl.ds(0, LAN)], ssem).wait()

    def group(j, carry):
        @pl.when(j > 0)
        def _():
            drain_scatters()
        gh = []
        for b in range(NB):
            off = (j * NB + b) * CH
            gh.append(pltpu.async_copy(x_hbm.at[src_f.at[pl.ds(off, CH)]],
                                       rows.at[b], gsem))
        for b in range(NB):
            gh[b].wait()
            off = (j * NB + b) * CH
            for k in range(nsub):
                iv = dst_f[pl.ds(off + k * LAN, LAN)]
                pltpu.async_copy(rows.at[b, pl.ds(k * LAN, LAN)],
                                 acc_sh.at[iv], ssem, add=True)
        return carry

    lax.fori_loop(0, NG, group, 0)
    drain_scatters()
    plsc.subcore_barrier()

    pltpu.sync_copy(acc_sh.at[pl.ds(s * RPD, RPD)],
                    acc_out.at[c, pl.ds(s * RPD, RPD)])


def _sc_deg_body(g2_hbm, zeros_hbm, ones_hbm, deg_out,
                 dst_f, ones_v, deg_sh, dsem):
    c = lax.axis_index("c")
    s = lax.axis_index("s")
    w = c * NS + s

    pltpu.sync_copy(zeros_hbm, deg_sh.at[pl.ds(s * RPD, RPD)])
    pltpu.sync_copy(ones_hbm, ones_v)
    pltpu.sync_copy(g2_hbm.at[1, pl.ds(w * EDG, EDG)], dst_f)
    plsc.subcore_barrier()

    def group(j, carry):
        sh = []
        for k in range(8):
            iv = dst_f[pl.ds((j * 8 + k) * LAN, LAN)]
            sh.append(pltpu.async_copy(ones_v, deg_sh.at[iv],
                                       dsem, add=True))
        for h in sh:
            h.wait()
        return carry

    lax.fori_loop(0, EDG // (8 * LAN), group, 0)
    plsc.subcore_barrier()

    pltpu.sync_copy(deg_sh.at[pl.ds(s * RPD, RPD)],
                    deg_out.at[c, pl.ds(s * RPD, RPD)])


@functools.lru_cache(maxsize=1)
def _sc_kernels():
    mesh = plsc.VectorSubcoreMesh(core_axis_name="c", subcore_axis_name="s")
    sc_agg = pl.kernel(
        _sc_agg_body,
        out_type=jax.ShapeDtypeStruct((NC, ND, D), jnp.float32),
        mesh=mesh,
        scratch_types=[
            pltpu.VMEM((EDG,), jnp.int32),
            pltpu.VMEM((EDG,), jnp.int32),
            pltpu.VMEM((NB, CH, D), jnp.float32),
            pltpu.VMEM_SHARED((ND, D), jnp.float32),
            pltpu.SemaphoreType.DMA,
            pltpu.SemaphoreType.DMA,
        ],
        name="sage_sc_agg",
    )
    sc_deg = pl.kernel(
        _sc_deg_body,
        out_type=jax.ShapeDtypeStruct((NC, ND, D), jnp.float32),
        mesh=mesh,
        scratch_types=[
            pltpu.VMEM((EDG,), jnp.int32),
            pltpu.VMEM((LAN, D), jnp.float32),
            pltpu.VMEM_SHARED((ND, D), jnp.float32),
            pltpu.SemaphoreType.DMA,
        ],
        name="sage_sc_deg",
    )
    return sc_agg, sc_deg


_ROWS = 200
_TCG = N // _ROWS


def _neigh(acc_ref, deg_ref, wn_ref, y_ref):
    agg = acc_ref[0] + acc_ref[1]
    dg = jnp.maximum(deg_ref[0, :, 0:1] + deg_ref[1, :, 0:1], 1.0)
    hn = agg / dg
    return y_ref[...] + jnp.dot(hn, wn_ref[...],
                                preferred_element_type=jnp.float32)


def _tca_body(h_ref, ws_ref, b_ref, o_ref):
    o_ref[...] = jnp.dot(h_ref[...], ws_ref[...],
                         preferred_element_type=jnp.float32) + b_ref[...]


def _tc1b_body(acc_ref, deg_ref, y_ref, wn_ref, o_ref):
    o_ref[...] = jnp.maximum(_neigh(acc_ref, deg_ref, wn_ref, y_ref), 0.0)


def _tc2b_body(acc_ref, deg_ref, y_ref, wn_ref,
               fcw_ref, fcb_ref, fc2w_ref, fc2b_ref, o_ref):
    t = jnp.maximum(_neigh(acc_ref, deg_ref, wn_ref, y_ref), 0.0)
    t = jnp.dot(t, fcw_ref[...], preferred_element_type=jnp.float32) + fcb_ref[...]
    t = jnp.maximum(t, 0.0)
    t = jnp.dot(t, fc2w_ref[...], preferred_element_type=jnp.float32) + fc2b_ref[...]
    o_ref[...] = jnp.maximum(t, 0.0)


def _tc3b_body(acc_ref, deg_ref, y_ref, wn_ref, o_ref):
    o_ref[...] = _neigh(acc_ref, deg_ref, wn_ref, y_ref)


_RSPEC = pl.BlockSpec((_ROWS, D), lambda i: (i, 0))
_WSPEC = pl.BlockSpec((D, D), lambda i: (0, 0))
_BSPEC = pl.BlockSpec((1, D), lambda i: (0, 0))
_ASPEC = pl.BlockSpec((NC, _ROWS, D), lambda i: (0, i, 0))


def _tc_call(body, specs):
    return pl.pallas_call(
        body,
        grid=(_TCG,),
        in_specs=specs,
        out_specs=_RSPEC,
        out_shape=jax.ShapeDtypeStruct((N, D), jnp.float32),
    )


def _tca(h, wsT, b):
    return _tc_call(_tca_body, [_RSPEC, _WSPEC, _BSPEC])(h, wsT, b)


def kernel(g, x, Ws1, Wn1, b1, Ws2, Wn2, b2, fcW, fcb, fc2W, fc2b, Ws3, Wn3, b3):
    npad = EP - E
    pad = jnp.stack([
        jnp.arange(npad, dtype=jnp.int32) % N,
        N + (jnp.arange(npad, dtype=jnp.int32) % (ND - N)),
    ])
    g2 = jnp.concatenate([g, pad], axis=1)
    zeros = jnp.zeros((RPD, D), jnp.float32)
    ones = jnp.ones((LAN, D), jnp.float32)

    sc_agg, sc_deg = _sc_kernels()
    deg = sc_deg(g2, zeros, ones)
    acc1 = sc_agg(x, g2, zeros)
    y1 = _tca(x, Ws1.T, b1.reshape(1, D))
    h1 = _tc_call(_tc1b_body, [_ASPEC, _ASPEC, _RSPEC, _WSPEC])(
        acc1, deg, y1, Wn1.T)

    acc2 = sc_agg(h1, g2, zeros)
    y2 = _tca(h1, Ws2.T, b2.reshape(1, D))
    h2 = _tc_call(_tc2b_body,
                  [_ASPEC, _ASPEC, _RSPEC, _WSPEC,
                   _WSPEC, _BSPEC, _WSPEC, _BSPEC])(
        acc2, deg, y2, Wn2.T,
        fcW.T, fcb.reshape(1, D), fc2W.T, fc2b.reshape(1, D))

    acc3 = sc_agg(h2, g2, zeros)
    y3 = _tca(h2, Ws3.T, b3.reshape(1, D))
    out = _tc_call(_tc3b_body, [_ASPEC, _ASPEC, _RSPEC, _WSPEC])(
        acc3, deg, y3, Wn3.T)
    return out


# speedup vs baseline: 1.0754x; 1.0167x over previous
import functools

import jax
import jax.numpy as jnp
from jax import lax
from jax.experimental import pallas as pl
from jax.experimental.pallas import tpu as pltpu
from jax.experimental.pallas import tpu_sc as plsc

N = 10000
E = 320000
EP = 327680
LAN = 16
D = 128
NC = 2
NS = 16
EDG = EP // (NC * NS)
CH = 16
NB = 10
NG = EDG // (NB * CH)
ND = 10240
RPD = ND // NS


def _sc_agg_body(x_hbm, g2_hbm, zeros_hbm, acc_out,
                 src_f, dst_f, rows, acc_sh, gsem, ssem):
    c = lax.axis_index("c")
    s = lax.axis_index("s")
    w = c * NS + s

    pltpu.sync_copy(zeros_hbm, acc_sh.at[pl.ds(s * RPD, RPD)])
    pltpu.sync_copy(g2_hbm.at[0, pl.ds(w * EDG, EDG)], src_f)
    pltpu.sync_copy(g2_hbm.at[1, pl.ds(w * EDG, EDG)], dst_f)
    plsc.subcore_barrier()

    nsub = CH // LAN

    def drain_scatters():
        for _ in range(NB * nsub):
            pltpu.make_async_copy(rows.at[0, pl.ds(0, LAN)],
                                  acc_sh.at[pl.ds(0, LAN)], ssem).wait()

    def group(j, carry):
        @pl.when(j > 0)
        def _():
            drain_scatters()
        gh = []
        for b in range(NB):
            off = (j * NB + b) * CH
            gh.append(pltpu.async_copy(x_hbm.at[src_f.at[pl.ds(off, CH)]],
                                       rows.at[b], gsem))
        for b in range(NB):
            gh[b].wait()
            off = (j * NB + b) * CH
            for k in range(nsub):
                iv = dst_f[pl.ds(off + k * LAN, LAN)]
                pltpu.async_copy(rows.at[b, pl.ds(k * LAN, LAN)],
                                 acc_sh.at[iv], ssem, add=True)
        return carry

    lax.fori_loop(0, NG, group, 0)
    drain_scatters()
    plsc.subcore_barrier()

    pltpu.sync_copy(acc_sh.at[pl.ds(s * RPD, RPD)],
                    acc_out.at[c, pl.ds(s * RPD, RPD)])


def _sc_deg_body(g2_hbm, zeros_hbm, ones_hbm, deg_out,
                 dst_f, ones_v, deg_sh, dsem):
    c = lax.axis_index("c")
    s = lax.axis_index("s")
    w = c * NS + s

    pltpu.sync_copy(zeros_hbm, deg_sh.at[pl.ds(s * RPD, RPD)])
    pltpu.sync_copy(ones_hbm, ones_v)
    pltpu.sync_copy(g2_hbm.at[1, pl.ds(w * EDG, EDG)], dst_f)
    plsc.subcore_barrier()

    def drain_deg():
        for _ in range(8):
            pltpu.make_async_copy(ones_v, deg_sh.at[pl.ds(0, LAN)],
                                  dsem).wait()

    def group(j, carry):
        @pl.when(j > 0)
        def _():
            drain_deg()
        for k in range(8):
            iv = dst_f[pl.ds((j * 8 + k) * LAN, LAN)]
            pltpu.async_copy(ones_v, deg_sh.at[iv], dsem, add=True)
        return carry

    lax.fori_loop(0, EDG // (8 * LAN), group, 0)
    drain_deg()
    plsc.subcore_barrier()

    pltpu.sync_copy(deg_sh.at[pl.ds(s * RPD, RPD)],
                    deg_out.at[c, pl.ds(s * RPD, RPD)])


@functools.lru_cache(maxsize=1)
def _sc_kernels():
    mesh = plsc.VectorSubcoreMesh(core_axis_name="c", subcore_axis_name="s")
    sc_agg = pl.kernel(
        _sc_agg_body,
        out_type=jax.ShapeDtypeStruct((NC, ND, D), jnp.float32),
        mesh=mesh,
        scratch_types=[
            pltpu.VMEM((EDG,), jnp.int32),
            pltpu.VMEM((EDG,), jnp.int32),
            pltpu.VMEM((NB, CH, D), jnp.float32),
            pltpu.VMEM_SHARED((ND, D), jnp.float32),
            pltpu.SemaphoreType.DMA,
            pltpu.SemaphoreType.DMA,
        ],
        name="sage_sc_agg",
    )
    sc_deg = pl.kernel(
        _sc_deg_body,
        out_type=jax.ShapeDtypeStruct((NC, ND, D), jnp.float32),
        mesh=mesh,
        scratch_types=[
            pltpu.VMEM((EDG,), jnp.int32),
            pltpu.VMEM((LAN, D), jnp.float32),
            pltpu.VMEM_SHARED((ND, D), jnp.float32),
            pltpu.SemaphoreType.DMA,
        ],
        name="sage_sc_deg",
    )
    return sc_agg, sc_deg


_ROWS = 200
_TCG = N // _ROWS


def _neigh(acc_ref, deg_ref, wn_ref, y_ref):
    agg = acc_ref[0] + acc_ref[1]
    dg = jnp.maximum(deg_ref[0, :, 0:1] + deg_ref[1, :, 0:1], 1.0)
    hn = agg / dg
    return y_ref[...] + jnp.dot(hn, wn_ref[...],
                                preferred_element_type=jnp.float32)


def _tca_body(h_ref, ws_ref, b_ref, o_ref):
    o_ref[...] = jnp.dot(h_ref[...], ws_ref[...],
                         preferred_element_type=jnp.float32) + b_ref[...]


def _tc1b_body(acc_ref, deg_ref, y_ref, wn_ref, o_ref):
    o_ref[...] = jnp.maximum(_neigh(acc_ref, deg_ref, wn_ref, y_ref), 0.0)


def _tc2b_body(acc_ref, deg_ref, y_ref, wn_ref,
               fcw_ref, fcb_ref, fc2w_ref, fc2b_ref, o_ref):
    t = jnp.maximum(_neigh(acc_ref, deg_ref, wn_ref, y_ref), 0.0)
    t = jnp.dot(t, fcw_ref[...], preferred_element_type=jnp.float32) + fcb_ref[...]
    t = jnp.maximum(t, 0.0)
    t = jnp.dot(t, fc2w_ref[...], preferred_element_type=jnp.float32) + fc2b_ref[...]
    o_ref[...] = jnp.maximum(t, 0.0)


def _tc3b_body(acc_ref, deg_ref, y_ref, wn_ref, o_ref):
    o_ref[...] = _neigh(acc_ref, deg_ref, wn_ref, y_ref)


_RSPEC = pl.BlockSpec((_ROWS, D), lambda i: (i, 0))
_WSPEC = pl.BlockSpec((D, D), lambda i: (0, 0))
_BSPEC = pl.BlockSpec((1, D), lambda i: (0, 0))
_ASPEC = pl.BlockSpec((NC, _ROWS, D), lambda i: (0, i, 0))


def _tc_call(body, specs):
    return pl.pallas_call(
        body,
        grid=(_TCG,),
        in_specs=specs,
        out_specs=_RSPEC,
        out_shape=jax.ShapeDtypeStruct((N, D), jnp.float32),
    )


def _tca(h, wsT, b):
    return _tc_call(_tca_body, [_RSPEC, _WSPEC, _BSPEC])(h, wsT, b)


def kernel(g, x, Ws1, Wn1, b1, Ws2, Wn2, b2, fcW, fcb, fc2W, fc2b, Ws3, Wn3, b3):
    npad = EP - E
    pad = jnp.stack([
        jnp.arange(npad, dtype=jnp.int32) % N,
        N + (jnp.arange(npad, dtype=jnp.int32) % (ND - N)),
    ])
    g2 = jnp.concatenate([g, pad], axis=1)
    zeros = jnp.zeros((RPD, D), jnp.float32)
    ones = jnp.ones((LAN, D), jnp.float32)

    sc_agg, sc_deg = _sc_kernels()
    deg = sc_deg(g2, zeros, ones)
    acc1 = sc_agg(x, g2, zeros)
    y1 = _tca(x, Ws1.T, b1.reshape(1, D))
    h1 = _tc_call(_tc1b_body, [_ASPEC, _ASPEC, _RSPEC, _WSPEC])(
        acc1, deg, y1, Wn1.T)

    acc2 = sc_agg(h1, g2, zeros)
    y2 = _tca(h1, Ws2.T, b2.reshape(1, D))
    h2 = _tc_call(_tc2b_body,
                  [_ASPEC, _ASPEC, _RSPEC, _WSPEC,
                   _WSPEC, _BSPEC, _WSPEC, _BSPEC])(
        acc2, deg, y2, Wn2.T,
        fcW.T, fcb.reshape(1, D), fc2W.T, fc2b.reshape(1, D))

    acc3 = sc_agg(h2, g2, zeros)
    y3 = _tca(h2, Ws3.T, b3.reshape(1, D))
    out = _tc_call(_tc3b_body, [_ASPEC, _ASPEC, _RSPEC, _WSPEC])(
        acc3, deg, y3, Wn3.T)
    return out
